# baseline (speedup 1.0000x reference)
import functools

import jax
import jax.numpy as jnp
from jax import lax
from jax.experimental import pallas as pl
from jax.experimental.pallas import tpu as pltpu
from jax.experimental.pallas import tpu_sc as plsc

B, H, W = 16, 512, 512
N = B * H * W
NC, NS, L = 2, 16, 16
NW = NC * NS

SC_IMGS = 4
TC_IMGS = B - SC_IMGS
ROWS_W = SC_IMGS * H // NW
CROWS = 32
NCHUNK = ROWS_W // CROWS

_C23 = float.fromhex('0x1.555556p-1')
_C25 = 0.4
_C27 = float.fromhex('0x1.24924ap-2')
_LMAX = 18.420680743952367


def _focal_body(logit_hbm, target_hbm, out_hbm, lbuf, tbuf, accv, sem0, sem1):
    wid = lax.axis_index("s") * NC + lax.axis_index("c")
    row0 = wid * ROWS_W
    b = row0 // H
    r0 = row0 % H
    sems = (sem0, sem1)

    def issue(g, slot):
        r = r0 + g * CROWS
        cl = pltpu.async_copy(logit_hbm.at[b, 0, pl.ds(r, CROWS), :],
                              lbuf.at[slot], sems[slot])
        ct = pltpu.async_copy(target_hbm.at[b, 0, pl.ds(r, CROWS), :],
                              tbuf.at[slot], sems[slot])
        return (cl, ct)

    def chunk_sum(slot, acc):
        def body(i, acc):
            r = i >> 5
            c = (i & 31) * L
            x = lbuf[slot, r, pl.ds(pl.multiple_of(c, L), L)]
            t = tbuf[slot, r, pl.ds(pl.multiple_of(c, L), L)]
            xb = lax.bitcast_convert_type(x, jnp.int32)
            wb = xb ^ (t << 31)
            w = lax.bitcast_convert_type(wb, jnp.float32)
            nu = lax.bitcast_convert_type(wb | jnp.int32(-2147483648),
                                          jnp.float32)
            a = jnp.exp(nu)
            one_a = a + 1.0
            two_a = a + 2.0
            pr = 1.0 / (one_a * two_a)
            dd = pr * two_a
            tt = (a * one_a) * pr
            omp = jnp.where(wb >= 0, dd, a * dd)
            t2 = tt * tt
            log1pa = tt * (2.0 + t2 * (_C23 + t2 * (_C25 + t2 * _C27)))
            sp = jnp.minimum(jnp.maximum(w, 0.0) + log1pa, _LMAX)
            return acc + (omp * omp) * sp
        return lax.fori_loop(0, CROWS * W // L, body, acc, unroll=4)

    pend = [issue(0, 0), None]
    acc = jnp.zeros((L,), jnp.float32)
    for g in range(NCHUNK):
        slot = g & 1
        if g + 1 < NCHUNK:
            pend[1 - slot] = issue(g + 1, 1 - slot)
        for cp in pend[slot]:
            cp.wait()
        acc = chunk_sum(slot, acc)
    accv[...] = acc
    pltpu.sync_copy(accv, out_hbm.at[wid])


_focal_sc = functools.partial(
    pl.kernel,
    out_type=jax.ShapeDtypeStruct((NW, L), jnp.float32),
    mesh=plsc.VectorSubcoreMesh(core_axis_name="c", subcore_axis_name="s"),
    compiler_params=pltpu.CompilerParams(use_tc_tiling_on_sc=True,
                                         disable_bounds_checks=True,
                                         disable_semaphore_checks=True,
                                         skip_device_barrier=True),
    scratch_types=[
        pltpu.VMEM((2, CROWS, W), jnp.float32),
        pltpu.VMEM((2, CROWS, W), jnp.int32),
        pltpu.VMEM((L,), jnp.float32),
        pltpu.SemaphoreType.DMA,
        pltpu.SemaphoreType.DMA,
    ],
)(_focal_body)


TC_ROWS = 256
TC_STEPS = TC_IMGS * H // TC_ROWS


def _focal_tc_body(x_ref, t_ref, o_ref):
    @pl.when(pl.program_id(0) == 0)
    def _init():
        o_ref[...] = jnp.zeros_like(o_ref)
    x = x_ref[0, 0]
    t = t_ref[0, 0]
    xb = lax.bitcast_convert_type(x, jnp.int32)
    wb = xb ^ (t << 31)
    w = lax.bitcast_convert_type(wb, jnp.float32)
    nu = lax.bitcast_convert_type(wb | jnp.int32(-2147483648),
                                  jnp.float32)
    omp = 0.5 + 0.5 * jnp.tanh(0.5 * w)
    sp = jnp.minimum(jnp.maximum(w, 0.0) + jnp.log1p(jnp.exp(nu)), _LMAX)
    loss = (omp * omp) * sp
    o_ref[...] += jnp.sum(loss.reshape(TC_ROWS // 8, 8, W), axis=0)


_ROWBLKS = H // TC_ROWS


def _tc_idx(i):
    return (SC_IMGS + i // _ROWBLKS, 0, i % _ROWBLKS, 0)


_focal_tc = pl.pallas_call(
    _focal_tc_body,
    grid=(TC_STEPS,),
    in_specs=[pl.BlockSpec((1, 1, TC_ROWS, W), _tc_idx),
              pl.BlockSpec((1, 1, TC_ROWS, W), _tc_idx)],
    out_specs=pl.BlockSpec((8, W), lambda i: (0, 0)),
    out_shape=jax.ShapeDtypeStruct((8, W), jnp.float32),
)


def kernel(logit, target):
    sc_part = _focal_sc(logit, target)
    tc_part = _focal_tc(logit, target)
    return (jnp.sum(sc_part) + jnp.sum(tc_part)) * jnp.float32(1.0 / N)


# speedup vs baseline: 1.0975x; 1.0975x over previous
import functools

import jax
import jax.numpy as jnp
from jax import lax
from jax.experimental import pallas as pl
from jax.experimental.pallas import tpu as pltpu
from jax.experimental.pallas import tpu_sc as plsc

B, H, W = 16, 512, 512
N = B * H * W
NC, NS, L = 2, 16, 16
NW = NC * NS

SC_IMGS = 4
TC_IMGS = B - SC_IMGS
ROWS_W = SC_IMGS * H // NW
CROWS = 32
NCHUNK = ROWS_W // CROWS

_C23 = float.fromhex('0x1.555556p-1')
_C25 = 0.4
_C27 = float.fromhex('0x1.24924ap-2')
_LMAX = 18.420680743952367


def _focal_body(logit_hbm, target_hbm, out_hbm, lbuf, tbuf, accv, sem0, sem1):
    wid = lax.axis_index("s") * NC + lax.axis_index("c")
    row0 = wid * ROWS_W
    b = row0 // H
    r0 = row0 % H
    sems = (sem0, sem1)

    def issue(g, slot):
        r = r0 + g * CROWS
        cl = pltpu.async_copy(logit_hbm.at[b, 0, pl.ds(r, CROWS), :],
                              lbuf.at[slot], sems[slot])
        ct = pltpu.async_copy(target_hbm.at[b, 0, pl.ds(r, CROWS), :],
                              tbuf.at[slot], sems[slot])
        return (cl, ct)

    def chunk_sum(slot, acc):
        def body(i, acc):
            r = i >> 5
            c = (i & 31) * L
            x = lbuf[slot, r, pl.ds(pl.multiple_of(c, L), L)]
            t = tbuf[slot, r, pl.ds(pl.multiple_of(c, L), L)]
            xb = lax.bitcast_convert_type(x, jnp.int32)
            wb = xb ^ (t << 31)
            w = lax.bitcast_convert_type(wb, jnp.float32)
            nu = lax.bitcast_convert_type(wb | jnp.int32(-2147483648),
                                          jnp.float32)
            a = jnp.exp(nu)
            one_a = a + 1.0
            two_a = a + 2.0
            pr = 1.0 / (one_a * two_a)
            dd = pr * two_a
            tt = (a * one_a) * pr
            omp = jnp.where(wb >= 0, dd, a * dd)
            t2 = tt * tt
            log1pa = tt * (2.0 + t2 * (_C23 + t2 * (_C25 + t2 * _C27)))
            sp = jnp.minimum(jnp.maximum(w, 0.0) + log1pa, _LMAX)
            return acc + (omp * omp) * sp
        return lax.fori_loop(0, CROWS * W // L, body, acc, unroll=4)

    pend = [issue(0, 0), None]
    acc = jnp.zeros((L,), jnp.float32)
    for g in range(NCHUNK):
        slot = g & 1
        if g + 1 < NCHUNK:
            pend[1 - slot] = issue(g + 1, 1 - slot)
        for cp in pend[slot]:
            cp.wait()
        acc = chunk_sum(slot, acc)
    accv[...] = acc
    pltpu.sync_copy(accv, out_hbm.at[wid])


_focal_sc = functools.partial(
    pl.kernel,
    out_type=jax.ShapeDtypeStruct((NW, L), jnp.float32),
    mesh=plsc.VectorSubcoreMesh(core_axis_name="c", subcore_axis_name="s"),
    compiler_params=pltpu.CompilerParams(use_tc_tiling_on_sc=True,
                                         disable_bounds_checks=True,
                                         disable_semaphore_checks=True,
                                         skip_device_barrier=True),
    scratch_types=[
        pltpu.VMEM((2, CROWS, W), jnp.float32),
        pltpu.VMEM((2, CROWS, W), jnp.int32),
        pltpu.VMEM((L,), jnp.float32),
        pltpu.SemaphoreType.DMA,
        pltpu.SemaphoreType.DMA,
    ],
)(_focal_body)


IMGB = 2
TC_ROWS = IMGB * H
TC_STEPS = TC_IMGS // IMGB


def _focal_tc_body(x_ref, t_ref, o_ref):
    @pl.when(pl.program_id(0) == 0)
    def _init():
        o_ref[...] = jnp.zeros_like(o_ref)
    x = x_ref[...].reshape(TC_ROWS, W)
    t = t_ref[...].reshape(TC_ROWS, W)
    xb = lax.bitcast_convert_type(x, jnp.int32)
    wb = xb ^ (t << 31)
    w = lax.bitcast_convert_type(wb, jnp.float32)
    nu = lax.bitcast_convert_type(wb | jnp.int32(-2147483648),
                                  jnp.float32)
    omp = 0.5 + 0.5 * jnp.tanh(0.5 * w)
    sp = jnp.minimum(jnp.maximum(w, 0.0) + jnp.log1p(jnp.exp(nu)), _LMAX)
    loss = (omp * omp) * sp
    o_ref[...] += jnp.sum(loss.reshape(TC_ROWS // 8, 8, W), axis=0)


def _tc_idx(i):
    return (SC_IMGS // IMGB + i, 0, 0, 0)


_focal_tc = pl.pallas_call(
    _focal_tc_body,
    grid=(TC_STEPS,),
    in_specs=[pl.BlockSpec((IMGB, 1, H, W), _tc_idx),
              pl.BlockSpec((IMGB, 1, H, W), _tc_idx)],
    out_specs=pl.BlockSpec((8, W), lambda i: (0, 0)),
    out_shape=jax.ShapeDtypeStruct((8, W), jnp.float32),
)


def kernel(logit, target):
    sc_part = _focal_sc(logit, target)
    tc_part = _focal_tc(logit, target)
    return (jnp.sum(sc_part) + jnp.sum(tc_part)) * jnp.float32(1.0 / N)


# speedup vs baseline: 1.1162x; 1.0170x over previous
import functools

import jax
import jax.numpy as jnp
from jax import lax
from jax.experimental import pallas as pl
from jax.experimental.pallas import tpu as pltpu
from jax.experimental.pallas import tpu_sc as plsc

B, H, W = 16, 512, 512
N = B * H * W
NC, NS, L = 2, 16, 16
NW = NC * NS

SC_IMGS = 4
TC_IMGS = B - SC_IMGS
ROWS_W = SC_IMGS * H // NW
CROWS = 32
NCHUNK = ROWS_W // CROWS

_C23 = float.fromhex('0x1.555556p-1')
_C25 = 0.4
_C27 = float.fromhex('0x1.24924ap-2')
_LMAX = 18.420680743952367


def _focal_body(logit_hbm, target_hbm, out_hbm, lbuf, tbuf, accv, sem0, sem1):
    wid = lax.axis_index("s") * NC + lax.axis_index("c")
    row0 = wid * ROWS_W
    b = row0 // H
    r0 = row0 % H
    sems = (sem0, sem1)

    def issue(g, slot):
        r = r0 + g * CROWS
        cl = pltpu.async_copy(logit_hbm.at[b, 0, pl.ds(r, CROWS), :],
                              lbuf.at[slot], sems[slot])
        ct = pltpu.async_copy(target_hbm.at[b, 0, pl.ds(r, CROWS), :],
                              tbuf.at[slot], sems[slot])
        return (cl, ct)

    def chunk_sum(slot, acc):
        def body(i, acc):
            r = i >> 5
            c = (i & 31) * L
            x = lbuf[slot, r, pl.ds(pl.multiple_of(c, L), L)]
            t = tbuf[slot, r, pl.ds(pl.multiple_of(c, L), L)]
            xb = lax.bitcast_convert_type(x, jnp.int32)
            wb = xb ^ (t << 31)
            w = lax.bitcast_convert_type(wb, jnp.float32)
            nu = lax.bitcast_convert_type(wb | jnp.int32(-2147483648),
                                          jnp.float32)
            a = jnp.exp(nu)
            one_a = a + 1.0
            two_a = a + 2.0
            pr = 1.0 / (one_a * two_a)
            dd = pr * two_a
            tt = (a * one_a) * pr
            omp = jnp.where(wb >= 0, dd, a * dd)
            t2 = tt * tt
            log1pa = tt * (2.0 + t2 * (_C23 + t2 * (_C25 + t2 * _C27)))
            sp = jnp.minimum(jnp.maximum(w, 0.0) + log1pa, _LMAX)
            return acc + (omp * omp) * sp
        return lax.fori_loop(0, CROWS * W // L, body, acc, unroll=2)

    pend = [issue(0, 0), None]
    acc = jnp.zeros((L,), jnp.float32)
    for g in range(NCHUNK):
        slot = g & 1
        if g + 1 < NCHUNK:
            pend[1 - slot] = issue(g + 1, 1 - slot)
        for cp in pend[slot]:
            cp.wait()
        acc = chunk_sum(slot, acc)
    accv[...] = acc
    pltpu.sync_copy(accv, out_hbm.at[wid])


_focal_sc = functools.partial(
    pl.kernel,
    out_type=jax.ShapeDtypeStruct((NW, L), jnp.float32),
    mesh=plsc.VectorSubcoreMesh(core_axis_name="c", subcore_axis_name="s"),
    compiler_params=pltpu.CompilerParams(use_tc_tiling_on_sc=True,
                                         disable_bounds_checks=True,
                                         disable_semaphore_checks=True,
                                         skip_device_barrier=True),
    scratch_types=[
        pltpu.VMEM((2, CROWS, W), jnp.float32),
        pltpu.VMEM((2, CROWS, W), jnp.int32),
        pltpu.VMEM((L,), jnp.float32),
        pltpu.SemaphoreType.DMA,
        pltpu.SemaphoreType.DMA,
    ],
)(_focal_body)


IMGB = 1
TC_ROWS = IMGB * H
TC_STEPS = TC_IMGS // IMGB


def _focal_tc_body(x_ref, t_ref, o_ref):
    @pl.when(pl.program_id(0) == 0)
    def _init():
        o_ref[...] = jnp.zeros_like(o_ref)
    x = x_ref[...].reshape(TC_ROWS, W)
    t = t_ref[...].reshape(TC_ROWS, W)
    xb = lax.bitcast_convert_type(x, jnp.int32)
    wb = xb ^ (t << 31)
    w = lax.bitcast_convert_type(wb, jnp.float32)
    nu = lax.bitcast_convert_type(wb | jnp.int32(-2147483648),
                                  jnp.float32)
    omp = 0.5 + 0.5 * jnp.tanh(0.5 * w)
    sp = jnp.minimum(jnp.maximum(w, 0.0) + jnp.log1p(jnp.exp(nu)), _LMAX)
    loss = (omp * omp) * sp
    o_ref[...] += jnp.sum(loss.reshape(TC_ROWS // 8, 8, W), axis=0)


def _tc_idx(i):
    return (SC_IMGS // IMGB + i, 0, 0, 0)


_focal_tc = pl.pallas_call(
    _focal_tc_body,
    grid=(TC_STEPS,),
    in_specs=[pl.BlockSpec((IMGB, 1, H, W), _tc_idx),
              pl.BlockSpec((IMGB, 1, H, W), _tc_idx)],
    out_specs=pl.BlockSpec((8, W), lambda i: (0, 0)),
    out_shape=jax.ShapeDtypeStruct((8, W), jnp.float32),
)


def kernel(logit, target):
    sc_part = _focal_sc(logit, target)
    tc_part = _focal_tc(logit, target)
    return (jnp.sum(sc_part) + jnp.sum(tc_part)) * jnp.float32(1.0 / N)
